# Initial kernel scaffold; baseline (speedup 1.0000x reference)
import jax
import jax.numpy as jnp
from jax.experimental import pallas as pl
from jax.experimental.pallas import tpu as pltpu
from jax.experimental.pallas import tpu_sc as plsc

T = 2048
H = 1024
E = 16
K = 2
NG = 4
GS = E // NG
I = 512
IS = 1024
RSF = 2.5
P = T * K
BM = 128
NBMAX = P // BM + E
PPAD = NBMAX * BM
BT = 256
SCW = 16

_NEG = jnp.float32(-jnp.inf)


def _router_body(scores_ref, bias_ref, pos_ref, w_ref, be_ref, na_ref):
    s_raw = scores_ref[...]
    s = s_raw + bias_ref[...]
    lane = jax.lax.broadcasted_iota(jnp.int32, (T, E), 1)

    gscores = []
    for g in range(NG):
        sub = jnp.where((lane // GS) == g, s, _NEG)
        m1 = jnp.max(sub, axis=1, keepdims=True)
        i1 = jnp.min(jnp.where(sub == m1, lane, E), axis=1, keepdims=True)
        m2 = jnp.max(jnp.where(lane == i1, _NEG, sub), axis=1, keepdims=True)
        gscores.append(m1 + m2)

    gm1 = gscores[0]
    for g in range(1, NG):
        gm1 = jnp.maximum(gm1, gscores[g])
    gi1 = jnp.full((T, 1), NG, jnp.int32)
    for g in reversed(range(NG)):
        gi1 = jnp.where(gscores[g] == gm1, g, gi1)
    gm2 = jnp.full((T, 1), _NEG)
    for g in range(NG):
        gm2 = jnp.maximum(gm2, jnp.where(gi1 == g, _NEG, gscores[g]))
    gi2 = jnp.full((T, 1), NG, jnp.int32)
    for g in reversed(range(NG)):
        gi2 = jnp.where((gscores[g] == gm2) & (gi1 != g), g, gi2)

    group_of_lane = lane // GS
    smask = (group_of_lane == gi1) | (group_of_lane == gi2)
    ms = jnp.where(smask, s, _NEG)

    m1 = jnp.max(ms, axis=1, keepdims=True)
    e1 = jnp.min(jnp.where(ms == m1, lane, E), axis=1, keepdims=True)
    ms2 = jnp.where(lane == e1, _NEG, ms)
    m2 = jnp.max(ms2, axis=1, keepdims=True)
    e2 = jnp.min(jnp.where(ms2 == m2, lane, E), axis=1, keepdims=True)
    oh1 = lane == e1
    oh2 = lane == e2

    w1 = jnp.sum(jnp.where(oh1, s_raw, 0.0), axis=1, keepdims=True)
    w2 = jnp.sum(jnp.where(oh2, s_raw, 0.0), axis=1, keepdims=True)
    denom = w1 + w2 + jnp.float32(1e-20)
    w1n = w1 / denom * jnp.float32(RSF)
    w2n = w2 / denom * jnp.float32(RSF)

    a1 = oh1.astype(jnp.float32)
    a2 = oh2.astype(jnp.float32)
    C = 128
    r = jax.lax.broadcasted_iota(jnp.int32, (C, C), 0)
    c = jax.lax.broadcasted_iota(jnp.int32, (C, C), 1)
    lexc = (r > c).astype(jnp.float32)
    off = jnp.zeros((1, E), jnp.float32)
    rank_chunks = []
    for a in (a1, a2):
        chunks = []
        for i in range(T // C):
            ch = a[i * C:(i + 1) * C, :]
            exc = jax.lax.dot_general(lexc, ch, (((1,), (0,)), ((), ())),
                                      preferred_element_type=jnp.float32)
            chunks.append(exc + off)
            off = off + jnp.sum(ch, axis=0, keepdims=True)
        rank_chunks.append(jnp.concatenate(chunks, axis=0))
    rank1, rank2 = rank_chunks
    counts = off

    nb = jnp.floor((counts + jnp.float32(BM - 1)) * jnp.float32(1.0 / BM))
    padded = nb * jnp.float32(BM)
    offs = []
    acc = jnp.zeros((1, 1), jnp.float32)
    for e in range(E):
        offs.append(acc)
        acc = acc + padded[:, e:e + 1]
    pad_off = jnp.concatenate(offs, axis=1)

    po1 = jnp.sum(jnp.where(oh1, pad_off, 0.0), axis=1, keepdims=True)
    po2 = jnp.sum(jnp.where(oh2, pad_off, 0.0), axis=1, keepdims=True)
    r1 = jnp.sum(jnp.where(oh1, rank1, 0.0), axis=1, keepdims=True)
    r2 = jnp.sum(jnp.where(oh2, rank2, 0.0), axis=1, keepdims=True)
    pos1 = (po1 + r1).astype(jnp.int32)
    pos2 = (po2 + r2).astype(jnp.int32)

    pos_ref[...] = jnp.concatenate([pos1, pos2], axis=1)
    w_ref[...] = jnp.concatenate([w1n, w2n], axis=1)

    bl = jax.lax.broadcasted_iota(jnp.float32, (1, NBMAX), 1)
    be = jnp.zeros((1, NBMAX), jnp.int32)
    na = jnp.zeros((1, 1), jnp.float32)
    last_e = jnp.zeros((1, 1), jnp.int32)
    for e in range(E):
        start = pad_off[:, e:e + 1] * jnp.float32(1.0 / BM)
        nbe = nb[:, e:e + 1]
        be = jnp.where((bl >= start) & (bl < start + nbe), e, be)
        na = na + nbe
        last_e = jnp.where(nbe > 0, e, last_e)
    be = jnp.where(bl >= na, last_e, be)
    be_ref[...] = be
    na_ref[...] = na.astype(jnp.int32)


def _router(scores, bias):
    return pl.pallas_call(
        _router_body,
        out_shape=(
            jax.ShapeDtypeStruct((T, K), jnp.int32),
            jax.ShapeDtypeStruct((T, K), jnp.float32),
            jax.ShapeDtypeStruct((1, NBMAX), jnp.int32),
            jax.ShapeDtypeStruct((1, 1), jnp.int32),
        ),
    )(scores, bias)


def _gmm_body(be_ref, na_ref, x_ref, wg_ref, wu_ref, wd_ref, y_ref):
    b = pl.program_id(0)

    @pl.when(b < na_ref[0])
    def _():
        x = x_ref[...]
        g = jax.lax.dot_general(x, wg_ref[0], (((1,), (1,)), ((), ())),
                                preferred_element_type=jnp.float32)
        u = jax.lax.dot_general(x, wu_ref[0], (((1,), (1,)), ((), ())),
                                preferred_element_type=jnp.float32)
        h = g * jax.nn.sigmoid(g) * u
        y_ref[...] = jax.lax.dot_general(h, wd_ref[0], (((1,), (1,)), ((), ())),
                                         preferred_element_type=jnp.float32)


def _gmm(be, na, x_sorted, W_gate, W_up, W_down):
    grid_spec = pltpu.PrefetchScalarGridSpec(
        num_scalar_prefetch=2,
        grid=(NBMAX,),
        in_specs=[
            pl.BlockSpec((BM, H), lambda b, be, na: (b, 0)),
            pl.BlockSpec((1, I, H), lambda b, be, na: (be[b], 0, 0)),
            pl.BlockSpec((1, I, H), lambda b, be, na: (be[b], 0, 0)),
            pl.BlockSpec((1, H, I), lambda b, be, na: (be[b], 0, 0)),
        ],
        out_specs=pl.BlockSpec((BM, H), lambda b, be, na: (b, 0)),
    )
    return pl.pallas_call(
        _gmm_body,
        grid_spec=grid_spec,
        out_shape=jax.ShapeDtypeStruct((PPAD, H), jnp.float32),
    )(be, na, x_sorted, W_gate, W_up, W_down)


def _shared_body(x_ref, wsg_ref, wsu_ref, wsd_ref, o_ref):
    x = x_ref[...]
    g = jax.lax.dot_general(x, wsg_ref[...], (((1,), (1,)), ((), ())),
                            preferred_element_type=jnp.float32)
    u = jax.lax.dot_general(x, wsu_ref[...], (((1,), (1,)), ((), ())),
                            preferred_element_type=jnp.float32)
    h = g * jax.nn.sigmoid(g) * u
    o_ref[...] = jax.lax.dot_general(h, wsd_ref[...], (((1,), (1,)), ((), ())),
                                     preferred_element_type=jnp.float32)


def _shared(x, Ws_gate, Ws_up, Ws_down):
    return pl.pallas_call(
        _shared_body,
        grid=(T // BT,),
        in_specs=[
            pl.BlockSpec((BT, H), lambda i: (i, 0)),
            pl.BlockSpec((IS, H), lambda i: (0, 0)),
            pl.BlockSpec((IS, H), lambda i: (0, 0)),
            pl.BlockSpec((H, IS), lambda i: (0, 0)),
        ],
        out_specs=pl.BlockSpec((BT, H), lambda i: (i, 0)),
        out_shape=jax.ShapeDtypeStruct((T, H), jnp.float32),
    )(x, Ws_gate, Ws_up, Ws_down)


def _sc_dispatch(x, pos0, pos1):
    mesh = plsc.VectorSubcoreMesh(core_axis_name="core", subcore_axis_name="subcore")

    @pl.kernel(out_type=jax.ShapeDtypeStruct((PPAD, H), jnp.float32), mesh=mesh)
    def k(x_hbm, p0_hbm, p1_hbm, o_hbm):
        def body(x_vmem, p0_vmem, p1_vmem):
            pltpu.sync_copy(x_vmem, o_hbm.at[p0_vmem.at[0]])
            pltpu.sync_copy(x_vmem, o_hbm.at[p1_vmem.at[0]])

        pltpu.emit_pipeline(
            body,
            grid=(T // SCW,),
            in_specs=[
                pl.BlockSpec((SCW, H), lambda i: (i, 0)),
                pl.BlockSpec((1, SCW), lambda i: (0, i)),
                pl.BlockSpec((1, SCW), lambda i: (0, i)),
            ],
            out_specs=[],
            core_axis_name=("core", "subcore"),
            dimension_semantics=(pltpu.PARALLEL,),
        )(x_hbm, p0_hbm, p1_hbm)

    return k(x, pos0, pos1)


def _sc_gather(y_sorted, pos_flat):
    mesh = plsc.VectorSubcoreMesh(core_axis_name="core", subcore_axis_name="subcore")

    @pl.kernel(out_type=jax.ShapeDtypeStruct((P, H), jnp.float32), mesh=mesh)
    def k(y_hbm, i_hbm, o_hbm):
        def body(i_vmem, o_vmem):
            pltpu.sync_copy(y_hbm.at[i_vmem.at[0]], o_vmem)

        pltpu.emit_pipeline(
            body,
            grid=(P // SCW,),
            in_specs=[pl.BlockSpec((1, SCW), lambda i: (0, i))],
            out_specs=[pl.BlockSpec((SCW, H), lambda i: (i, 0))],
            core_axis_name=("core", "subcore"),
            dimension_semantics=(pltpu.PARALLEL,),
        )(i_hbm, o_hbm)

    return k(y_sorted, pos_flat)


def _epilogue_body(sh_ref, yp_ref, w_ref, o_ref):
    yp = yp_ref[...]
    w = w_ref[...]
    o_ref[...] = (sh_ref[...]
                  + w[:, 0:1] * yp[:, :H]
                  + w[:, 1:2] * yp[:, H:])


def _epilogue(shared, y_pair2, w):
    return pl.pallas_call(
        _epilogue_body,
        grid=(T // BT,),
        in_specs=[
            pl.BlockSpec((BT, H), lambda i: (i, 0)),
            pl.BlockSpec((BT, K * H), lambda i: (i, 0)),
            pl.BlockSpec((BT, K), lambda i: (i, 0)),
        ],
        out_specs=pl.BlockSpec((BT, H), lambda i: (i, 0)),
        out_shape=jax.ShapeDtypeStruct((T, H), jnp.float32),
    )(shared, y_pair2, w)


def kernel(hidden_states, Wg_router, bias_corr, W_gate, W_up, W_down,
           Ws_gate, Ws_up, Ws_down):
    x = hidden_states
    logits = jnp.dot(x, Wg_router.T).astype(jnp.float32)
    scores = jax.nn.sigmoid(logits)

    pos, w, be, na = _router(scores, bias_corr.reshape(1, E))
    pos0 = pos[:, 0].reshape(1, T)
    pos1 = pos[:, 1].reshape(1, T)
    pos_flat = pos.reshape(1, P)

    x_sorted = _sc_dispatch(x, pos0, pos1)
    shared = _shared(x, Ws_gate, Ws_up, Ws_down)
    y_sorted = _gmm(be.reshape(NBMAX), na.reshape(1), x_sorted,
                    W_gate, W_up, W_down)
    y_pair = _sc_gather(y_sorted, pos_flat)
    return _epilogue(shared, y_pair.reshape(T, K * H), w)


# speedup vs baseline: 1.5993x; 1.5993x over previous
import jax
import jax.numpy as jnp
from jax.experimental import pallas as pl
from jax.experimental.pallas import tpu as pltpu
from jax.experimental.pallas import tpu_sc as plsc

T = 2048
H = 1024
E = 16
K = 2
NG = 4
GS = E // NG
I = 512
IS = 1024
RSF = 2.5
P = T * K
BM = 128
NBMAX = P // BM + E
PPAD = NBMAX * BM
BT = 256
SR = 4
HS = H // SR
SCW = 128

_NEG = float("-inf")


def _router_body(scores_ref, bias_ref, pos_ref, w_ref, be_ref, na_ref):
    s_raw = scores_ref[...]
    s = s_raw + bias_ref[...]
    lane = jax.lax.broadcasted_iota(jnp.int32, (T, E), 1)

    gscores = []
    for g in range(NG):
        sub = jnp.where((lane // GS) == g, s, _NEG)
        m1 = jnp.max(sub, axis=1, keepdims=True)
        i1 = jnp.min(jnp.where(sub == m1, lane, E), axis=1, keepdims=True)
        m2 = jnp.max(jnp.where(lane == i1, _NEG, sub), axis=1, keepdims=True)
        gscores.append(m1 + m2)

    gm1 = gscores[0]
    for g in range(1, NG):
        gm1 = jnp.maximum(gm1, gscores[g])
    gi1 = jnp.full((T, 1), NG, jnp.int32)
    for g in reversed(range(NG)):
        gi1 = jnp.where(gscores[g] == gm1, g, gi1)
    gm2 = jnp.full((T, 1), _NEG)
    for g in range(NG):
        gm2 = jnp.maximum(gm2, jnp.where(gi1 == g, _NEG, gscores[g]))
    gi2 = jnp.full((T, 1), NG, jnp.int32)
    for g in reversed(range(NG)):
        gi2 = jnp.where((gscores[g] == gm2) & (gi1 != g), g, gi2)

    group_of_lane = lane // GS
    smask = (group_of_lane == gi1) | (group_of_lane == gi2)
    ms = jnp.where(smask, s, _NEG)

    m1 = jnp.max(ms, axis=1, keepdims=True)
    e1 = jnp.min(jnp.where(ms == m1, lane, E), axis=1, keepdims=True)
    ms2 = jnp.where(lane == e1, _NEG, ms)
    m2 = jnp.max(ms2, axis=1, keepdims=True)
    e2 = jnp.min(jnp.where(ms2 == m2, lane, E), axis=1, keepdims=True)
    oh1 = lane == e1
    oh2 = lane == e2

    w1 = jnp.sum(jnp.where(oh1, s_raw, 0.0), axis=1, keepdims=True)
    w2 = jnp.sum(jnp.where(oh2, s_raw, 0.0), axis=1, keepdims=True)
    denom = w1 + w2 + jnp.float32(1e-20)
    w1n = w1 / denom * jnp.float32(RSF)
    w2n = w2 / denom * jnp.float32(RSF)

    a1 = oh1.astype(jnp.float32)
    a2 = oh2.astype(jnp.float32)
    C = 128
    r = jax.lax.broadcasted_iota(jnp.int32, (C, C), 0)
    c = jax.lax.broadcasted_iota(jnp.int32, (C, C), 1)
    lexc = (r > c).astype(jnp.float32)
    off = jnp.zeros((1, E), jnp.float32)
    rank_chunks = []
    for a in (a1, a2):
        chunks = []
        for i in range(T // C):
            ch = a[i * C:(i + 1) * C, :]
            exc = jax.lax.dot_general(lexc, ch, (((1,), (0,)), ((), ())),
                                      preferred_element_type=jnp.float32)
            chunks.append(exc + off)
            off = off + jnp.sum(ch, axis=0, keepdims=True)
        rank_chunks.append(jnp.concatenate(chunks, axis=0))
    rank1, rank2 = rank_chunks
    counts = off

    nb = jnp.floor((counts + jnp.float32(BM - 1)) * jnp.float32(1.0 / BM))
    padded = nb * jnp.float32(BM)
    offs = []
    acc = jnp.zeros((1, 1), jnp.float32)
    for e in range(E):
        offs.append(acc)
        acc = acc + padded[:, e:e + 1]
    pad_off = jnp.concatenate(offs, axis=1)

    po1 = jnp.sum(jnp.where(oh1, pad_off, 0.0), axis=1, keepdims=True)
    po2 = jnp.sum(jnp.where(oh2, pad_off, 0.0), axis=1, keepdims=True)
    r1 = jnp.sum(jnp.where(oh1, rank1, 0.0), axis=1, keepdims=True)
    r2 = jnp.sum(jnp.where(oh2, rank2, 0.0), axis=1, keepdims=True)
    pos1 = (po1 + r1).astype(jnp.int32)
    pos2 = (po2 + r2).astype(jnp.int32)

    pos_ref[...] = jnp.concatenate([pos1, pos2], axis=1)
    w_ref[...] = jnp.concatenate([w1n, w2n], axis=1)

    bl = jax.lax.broadcasted_iota(jnp.int32, (1, NBMAX), 1).astype(jnp.float32)
    be = jnp.zeros((1, NBMAX), jnp.int32)
    na = jnp.zeros((1, 1), jnp.float32)
    last_e = jnp.zeros((1, 1), jnp.int32)
    for e in range(E):
        start = pad_off[:, e:e + 1] * jnp.float32(1.0 / BM)
        nbe = nb[:, e:e + 1]
        be = jnp.where((bl >= start) & (bl < start + nbe), e, be)
        na = na + nbe
        last_e = jnp.where(nbe > 0, e, last_e)
    be = jnp.where(bl >= na, last_e, be)
    be_ref[...] = be
    na_ref[...] = na.astype(jnp.int32)


def _router(scores, bias):
    return pl.pallas_call(
        _router_body,
        out_shape=(
            jax.ShapeDtypeStruct((T, K), jnp.int32),
            jax.ShapeDtypeStruct((T, K), jnp.float32),
            jax.ShapeDtypeStruct((1, NBMAX), jnp.int32),
            jax.ShapeDtypeStruct((1, 1), jnp.int32),
        ),
    )(scores, bias)


def _gmm_body(be_ref, na_ref, x_ref, wg_ref, wu_ref, wd_ref, y_ref):
    b = pl.program_id(0)

    @pl.when(b < na_ref[0])
    def _():
        x = x_ref[...]
        g = jax.lax.dot_general(x, wg_ref[0], (((1,), (1,)), ((), ())),
                                preferred_element_type=jnp.float32)
        u = jax.lax.dot_general(x, wu_ref[0], (((1,), (1,)), ((), ())),
                                preferred_element_type=jnp.float32)
        h = g * jax.nn.sigmoid(g) * u
        y_ref[...] = jax.lax.dot_general(h, wd_ref[0], (((1,), (1,)), ((), ())),
                                         preferred_element_type=jnp.float32)


def _gmm(be, na, x_sorted, W_gate, W_up, W_down):
    grid_spec = pltpu.PrefetchScalarGridSpec(
        num_scalar_prefetch=2,
        grid=(NBMAX,),
        in_specs=[
            pl.BlockSpec((BM, H), lambda b, be, na: (b, 0)),
            pl.BlockSpec((1, I, H), lambda b, be, na: (be[b], 0, 0)),
            pl.BlockSpec((1, I, H), lambda b, be, na: (be[b], 0, 0)),
            pl.BlockSpec((1, H, I), lambda b, be, na: (be[b], 0, 0)),
        ],
        out_specs=pl.BlockSpec((BM, H), lambda b, be, na: (b, 0)),
    )
    return pl.pallas_call(
        _gmm_body,
        grid_spec=grid_spec,
        out_shape=jax.ShapeDtypeStruct((PPAD, H), jnp.float32),
    )(be, na, x_sorted, W_gate, W_up, W_down)


def _shared_body(x_ref, wsg_ref, wsu_ref, wsd_ref, o_ref):
    x = x_ref[...]
    g = jax.lax.dot_general(x, wsg_ref[...], (((1,), (1,)), ((), ())),
                            preferred_element_type=jnp.float32)
    u = jax.lax.dot_general(x, wsu_ref[...], (((1,), (1,)), ((), ())),
                            preferred_element_type=jnp.float32)
    h = g * jax.nn.sigmoid(g) * u
    o_ref[...] = jax.lax.dot_general(h, wsd_ref[...], (((1,), (1,)), ((), ())),
                                     preferred_element_type=jnp.float32)


def _shared(x, Ws_gate, Ws_up, Ws_down):
    return pl.pallas_call(
        _shared_body,
        grid=(T // BT,),
        in_specs=[
            pl.BlockSpec((BT, H), lambda i: (i, 0)),
            pl.BlockSpec((IS, H), lambda i: (0, 0)),
            pl.BlockSpec((IS, H), lambda i: (0, 0)),
            pl.BlockSpec((H, IS), lambda i: (0, 0)),
        ],
        out_specs=pl.BlockSpec((BT, H), lambda i: (i, 0)),
        out_shape=jax.ShapeDtypeStruct((T, H), jnp.float32),
    )(x, Ws_gate, Ws_up, Ws_down)


def _sc_dispatch(x4, dst0, dst1):
    mesh = plsc.VectorSubcoreMesh(core_axis_name="core", subcore_axis_name="subcore")

    @pl.kernel(out_type=jax.ShapeDtypeStruct((PPAD * SR, HS), jnp.float32),
               mesh=mesh)
    def k(x_hbm, p0_hbm, p1_hbm, o_hbm):
        def body(x_vmem, p0_vmem, p1_vmem):
            pltpu.sync_copy(x_vmem, o_hbm.at[p0_vmem.at[0]])
            pltpu.sync_copy(x_vmem, o_hbm.at[p1_vmem.at[0]])

        pltpu.emit_pipeline(
            body,
            grid=(T * SR // SCW,),
            in_specs=[
                pl.BlockSpec((SCW, HS), lambda i: (i, 0)),
                pl.BlockSpec((1, SCW), lambda i: (0, i)),
                pl.BlockSpec((1, SCW), lambda i: (0, i)),
            ],
            out_specs=[],
            core_axis_name=("core", "subcore"),
            dimension_semantics=(pltpu.PARALLEL,),
        )(x_hbm, p0_hbm, p1_hbm)

    return k(x4, dst0, dst1)


def _sc_gather(y4, gidx):
    mesh = plsc.VectorSubcoreMesh(core_axis_name="core", subcore_axis_name="subcore")

    @pl.kernel(out_type=jax.ShapeDtypeStruct((P * SR, HS), jnp.float32),
               mesh=mesh)
    def k(y_hbm, i_hbm, o_hbm):
        def body(i_vmem, o_vmem):
            pltpu.sync_copy(y_hbm.at[i_vmem.at[0]], o_vmem)

        pltpu.emit_pipeline(
            body,
            grid=(P * SR // SCW,),
            in_specs=[pl.BlockSpec((1, SCW), lambda i: (0, i))],
            out_specs=[pl.BlockSpec((SCW, HS), lambda i: (i, 0))],
            core_axis_name=("core", "subcore"),
            dimension_semantics=(pltpu.PARALLEL,),
        )(i_hbm, o_hbm)

    return k(y4, gidx)


def _epilogue_body(sh_ref, yp_ref, w_ref, o_ref):
    yp = yp_ref[...]
    w = w_ref[...]
    o_ref[...] = (sh_ref[...]
                  + w[:, 0:1] * yp[:, :H]
                  + w[:, 1:2] * yp[:, H:])


def _epilogue(shared, y_pair2, w):
    return pl.pallas_call(
        _epilogue_body,
        grid=(T // BT,),
        in_specs=[
            pl.BlockSpec((BT, H), lambda i: (i, 0)),
            pl.BlockSpec((BT, K * H), lambda i: (i, 0)),
            pl.BlockSpec((BT, K), lambda i: (i, 0)),
        ],
        out_specs=pl.BlockSpec((BT, H), lambda i: (i, 0)),
        out_shape=jax.ShapeDtypeStruct((T, H), jnp.float32),
    )(shared, y_pair2, w)


def kernel(hidden_states, Wg_router, bias_corr, W_gate, W_up, W_down,
           Ws_gate, Ws_up, Ws_down):
    x = hidden_states
    logits = jnp.dot(x, Wg_router.T).astype(jnp.float32)
    scores = jax.nn.sigmoid(logits)

    pos, w, be, na = _router(scores, bias_corr.reshape(1, E))
    j = jnp.arange(SR, dtype=jnp.int32)[None, :]
    dst0 = (pos[:, 0:1] * SR + j).reshape(1, T * SR)
    dst1 = (pos[:, 1:2] * SR + j).reshape(1, T * SR)
    gidx = (pos.reshape(P, 1) * SR + j).reshape(1, P * SR)

    x_sorted4 = _sc_dispatch(x.reshape(T * SR, HS), dst0, dst1)
    shared = _shared(x, Ws_gate, Ws_up, Ws_down)
    y_sorted = _gmm(be.reshape(NBMAX), na.reshape(1),
                    x_sorted4.reshape(PPAD, H), W_gate, W_up, W_down)
    y_pair4 = _sc_gather(y_sorted.reshape(PPAD * SR, HS), gidx)
    return _epilogue(shared, y_pair4.reshape(T, K * H), w)


# speedup vs baseline: 2.4237x; 1.5155x over previous
import jax
import jax.numpy as jnp
from jax.experimental import pallas as pl
from jax.experimental.pallas import tpu as pltpu
from jax.experimental.pallas import tpu_sc as plsc

T = 2048
H = 1024
E = 16
K = 2
NG = 4
GS = E // NG
I = 512
IS = 1024
RSF = 2.5
P = T * K
BM = 128
NBMAX = P // BM + E
PPAD = NBMAX * BM
BT = 256
SR = 4
HS = H // SR
HSW = HS // 2
SCW = 128

_NEG = float("-inf")


def _router_body(scores_ref, bias_ref, pos_ref, w_ref, be_ref, na_ref):
    s_raw = scores_ref[...]
    s = s_raw + bias_ref[...]
    lane = jax.lax.broadcasted_iota(jnp.int32, (T, E), 1)

    gscores = []
    for g in range(NG):
        sub = jnp.where((lane // GS) == g, s, _NEG)
        m1 = jnp.max(sub, axis=1, keepdims=True)
        i1 = jnp.min(jnp.where(sub == m1, lane, E), axis=1, keepdims=True)
        m2 = jnp.max(jnp.where(lane == i1, _NEG, sub), axis=1, keepdims=True)
        gscores.append(m1 + m2)

    gm1 = gscores[0]
    for g in range(1, NG):
        gm1 = jnp.maximum(gm1, gscores[g])
    gi1 = jnp.full((T, 1), NG, jnp.int32)
    for g in reversed(range(NG)):
        gi1 = jnp.where(gscores[g] == gm1, g, gi1)
    gm2 = jnp.full((T, 1), _NEG)
    for g in range(NG):
        gm2 = jnp.maximum(gm2, jnp.where(gi1 == g, _NEG, gscores[g]))
    gi2 = jnp.full((T, 1), NG, jnp.int32)
    for g in reversed(range(NG)):
        gi2 = jnp.where((gscores[g] == gm2) & (gi1 != g), g, gi2)

    group_of_lane = lane // GS
    smask = (group_of_lane == gi1) | (group_of_lane == gi2)
    ms = jnp.where(smask, s, _NEG)

    m1 = jnp.max(ms, axis=1, keepdims=True)
    e1 = jnp.min(jnp.where(ms == m1, lane, E), axis=1, keepdims=True)
    ms2 = jnp.where(lane == e1, _NEG, ms)
    m2 = jnp.max(ms2, axis=1, keepdims=True)
    e2 = jnp.min(jnp.where(ms2 == m2, lane, E), axis=1, keepdims=True)
    oh1 = lane == e1
    oh2 = lane == e2

    w1 = jnp.sum(jnp.where(oh1, s_raw, 0.0), axis=1, keepdims=True)
    w2 = jnp.sum(jnp.where(oh2, s_raw, 0.0), axis=1, keepdims=True)
    denom = w1 + w2 + jnp.float32(1e-20)
    w1n = w1 / denom * jnp.float32(RSF)
    w2n = w2 / denom * jnp.float32(RSF)

    a1 = oh1.astype(jnp.float32)
    a2 = oh2.astype(jnp.float32)
    C = 128
    r = jax.lax.broadcasted_iota(jnp.int32, (C, C), 0)
    c = jax.lax.broadcasted_iota(jnp.int32, (C, C), 1)
    lexc = (r > c).astype(jnp.float32)
    off = jnp.zeros((1, E), jnp.float32)
    rank_chunks = []
    for a in (a1, a2):
        chunks = []
        for i in range(T // C):
            ch = a[i * C:(i + 1) * C, :]
            exc = jax.lax.dot_general(lexc, ch, (((1,), (0,)), ((), ())),
                                      preferred_element_type=jnp.float32)
            chunks.append(exc + off)
            off = off + jnp.sum(ch, axis=0, keepdims=True)
        rank_chunks.append(jnp.concatenate(chunks, axis=0))
    rank1, rank2 = rank_chunks
    counts = off

    nb = jnp.floor((counts + jnp.float32(BM - 1)) * jnp.float32(1.0 / BM))
    padded = nb * jnp.float32(BM)
    offs = []
    acc = jnp.zeros((1, 1), jnp.float32)
    for e in range(E):
        offs.append(acc)
        acc = acc + padded[:, e:e + 1]
    pad_off = jnp.concatenate(offs, axis=1)

    po1 = jnp.sum(jnp.where(oh1, pad_off, 0.0), axis=1, keepdims=True)
    po2 = jnp.sum(jnp.where(oh2, pad_off, 0.0), axis=1, keepdims=True)
    r1 = jnp.sum(jnp.where(oh1, rank1, 0.0), axis=1, keepdims=True)
    r2 = jnp.sum(jnp.where(oh2, rank2, 0.0), axis=1, keepdims=True)
    pos1 = (po1 + r1).astype(jnp.int32)
    pos2 = (po2 + r2).astype(jnp.int32)

    pos_ref[...] = jnp.concatenate([pos1, pos2], axis=1)
    w_ref[...] = jnp.concatenate([w1n, w2n], axis=1)

    bl = jax.lax.broadcasted_iota(jnp.int32, (1, NBMAX), 1).astype(jnp.float32)
    be = jnp.zeros((1, NBMAX), jnp.int32)
    na = jnp.zeros((1, 1), jnp.float32)
    last_e = jnp.zeros((1, 1), jnp.int32)
    for e in range(E):
        start = pad_off[:, e:e + 1] * jnp.float32(1.0 / BM)
        nbe = nb[:, e:e + 1]
        be = jnp.where((bl >= start) & (bl < start + nbe), e, be)
        na = na + nbe
        last_e = jnp.where(nbe > 0, e, last_e)
    be = jnp.where(bl >= na, last_e, be)
    be_ref[...] = be
    na_ref[...] = na.astype(jnp.int32)


def _router(scores, bias):
    return pl.pallas_call(
        _router_body,
        out_shape=(
            jax.ShapeDtypeStruct((T, K), jnp.int32),
            jax.ShapeDtypeStruct((T, K), jnp.float32),
            jax.ShapeDtypeStruct((1, NBMAX), jnp.int32),
            jax.ShapeDtypeStruct((1, 1), jnp.int32),
        ),
    )(scores, bias)


def _gmm_body(be_ref, na_ref, x_ref, wg_ref, wu_ref, wd_ref, y_ref):
    b = pl.program_id(0)

    @pl.when(b < na_ref[0])
    def _():
        x = x_ref[...].astype(jnp.bfloat16)
        wg = wg_ref[0].astype(jnp.bfloat16)
        wu = wu_ref[0].astype(jnp.bfloat16)
        wd = wd_ref[0].astype(jnp.bfloat16)
        g = jax.lax.dot_general(x, wg, (((1,), (1,)), ((), ())),
                                preferred_element_type=jnp.float32)
        u = jax.lax.dot_general(x, wu, (((1,), (1,)), ((), ())),
                                preferred_element_type=jnp.float32)
        h = (g * jax.nn.sigmoid(g) * u).astype(jnp.bfloat16)
        y_ref[...] = jax.lax.dot_general(h, wd, (((1,), (1,)), ((), ())),
                                         preferred_element_type=jnp.float32)


def _gmm(be, na, x_sorted, W_gate, W_up, W_down):
    def _xy_idx(b, be, na):
        return (jnp.minimum(b, na[0] - 1), 0)

    def _w_idx(b, be, na):
        return (be[b], 0, 0)

    grid_spec = pltpu.PrefetchScalarGridSpec(
        num_scalar_prefetch=2,
        grid=(NBMAX,),
        in_specs=[
            pl.BlockSpec((BM, H), _xy_idx),
            pl.BlockSpec((1, I, H), _w_idx),
            pl.BlockSpec((1, I, H), _w_idx),
            pl.BlockSpec((1, H, I), _w_idx),
        ],
        out_specs=pl.BlockSpec((BM, H), _xy_idx),
    )
    return pl.pallas_call(
        _gmm_body,
        grid_spec=grid_spec,
        out_shape=jax.ShapeDtypeStruct((PPAD, H), jnp.float32),
    )(be, na, x_sorted, W_gate, W_up, W_down)


def _shared_body(x_ref, wsg_ref, wsu_ref, wsd_ref, o_ref):
    x = x_ref[...].astype(jnp.bfloat16)
    wsg = wsg_ref[...].astype(jnp.bfloat16)
    wsu = wsu_ref[...].astype(jnp.bfloat16)
    wsd = wsd_ref[...].astype(jnp.bfloat16)
    g = jax.lax.dot_general(x, wsg, (((1,), (1,)), ((), ())),
                            preferred_element_type=jnp.float32)
    u = jax.lax.dot_general(x, wsu, (((1,), (1,)), ((), ())),
                            preferred_element_type=jnp.float32)
    h = (g * jax.nn.sigmoid(g) * u).astype(jnp.bfloat16)
    o_ref[...] = jax.lax.dot_general(h, wsd, (((1,), (1,)), ((), ())),
                                     preferred_element_type=jnp.float32)


def _shared(x, Ws_gate, Ws_up, Ws_down):
    return pl.pallas_call(
        _shared_body,
        grid=(T // BT,),
        in_specs=[
            pl.BlockSpec((BT, H), lambda i: (i, 0)),
            pl.BlockSpec((IS, H), lambda i: (0, 0)),
            pl.BlockSpec((IS, H), lambda i: (0, 0)),
            pl.BlockSpec((H, IS), lambda i: (0, 0)),
        ],
        out_specs=pl.BlockSpec((BT, H), lambda i: (i, 0)),
        out_shape=jax.ShapeDtypeStruct((T, H), jnp.float32),
    )(x, Ws_gate, Ws_up, Ws_down)


DR = 32
GR = 16
IW = 128


def _sc_dispatch(x, pos0p, pos1p):
    mesh = plsc.VectorSubcoreMesh(core_axis_name="core", subcore_axis_name="subcore")

    @pl.kernel(out_type=jax.ShapeDtypeStruct((PPAD, H), jnp.float32),
               mesh=mesh)
    def k(x_hbm, p0_hbm, p1_hbm, o_hbm):
        def body(x_vmem, p0_vmem, p1_vmem):
            pltpu.sync_copy(x_vmem, o_hbm.at[p0_vmem.at[0, pl.ds(0, DR)]])
            pltpu.sync_copy(x_vmem, o_hbm.at[p1_vmem.at[0, pl.ds(0, DR)]])

        pltpu.emit_pipeline(
            body,
            grid=(T // DR,),
            in_specs=[
                pl.BlockSpec((DR, H), lambda i: (i, 0)),
                pl.BlockSpec((1, IW), lambda i: (i, 0)),
                pl.BlockSpec((1, IW), lambda i: (i, 0)),
            ],
            out_specs=[],
            core_axis_name=("core", "subcore"),
            dimension_semantics=(pltpu.PARALLEL,),
        )(x_hbm, p0_hbm, p1_hbm)

    return k(x, pos0p, pos1p)


def _sc_gather(y_sorted, pos0p, pos1p):
    mesh = plsc.VectorSubcoreMesh(core_axis_name="core", subcore_axis_name="subcore")

    @pl.kernel(out_type=(jax.ShapeDtypeStruct((T, H), jnp.float32),
                         jax.ShapeDtypeStruct((T, H), jnp.float32)),
               mesh=mesh)
    def k(y_hbm, p0_hbm, p1_hbm, o0_hbm, o1_hbm):
        def body(p0_vmem, p1_vmem, o0_vmem, o1_vmem):
            pltpu.sync_copy(y_hbm.at[p0_vmem.at[0, pl.ds(0, GR)]], o0_vmem)
            pltpu.sync_copy(y_hbm.at[p1_vmem.at[0, pl.ds(0, GR)]], o1_vmem)

        pltpu.emit_pipeline(
            body,
            grid=(T // GR,),
            in_specs=[
                pl.BlockSpec((1, IW), lambda i: (i, 0)),
                pl.BlockSpec((1, IW), lambda i: (i, 0)),
            ],
            out_specs=[
                pl.BlockSpec((GR, H), lambda i: (i, 0)),
                pl.BlockSpec((GR, H), lambda i: (i, 0)),
            ],
            core_axis_name=("core", "subcore"),
            dimension_semantics=(pltpu.PARALLEL,),
        )(p0_hbm, p1_hbm, o0_hbm, o1_hbm)

    return k(y_sorted, pos0p, pos1p)


def _epilogue_body(sh_ref, y0_ref, y1_ref, w_ref, o_ref):
    w = w_ref[...]
    o_ref[...] = (sh_ref[...]
                  + w[:, 0:1] * y0_ref[...]
                  + w[:, 1:2] * y1_ref[...])


def _epilogue(shared, y0, y1, w):
    return pl.pallas_call(
        _epilogue_body,
        grid=(T // BT,),
        in_specs=[
            pl.BlockSpec((BT, H), lambda i: (i, 0)),
            pl.BlockSpec((BT, H), lambda i: (i, 0)),
            pl.BlockSpec((BT, H), lambda i: (i, 0)),
            pl.BlockSpec((BT, K), lambda i: (i, 0)),
        ],
        out_specs=pl.BlockSpec((BT, H), lambda i: (i, 0)),
        out_shape=jax.ShapeDtypeStruct((T, H), jnp.float32),
    )(shared, y0, y1, w)


def kernel(hidden_states, Wg_router, bias_corr, W_gate, W_up, W_down,
           Ws_gate, Ws_up, Ws_down):
    x = hidden_states
    logits = jnp.dot(x, Wg_router.T).astype(jnp.float32)
    scores = jax.nn.sigmoid(logits)

    pos, w, be, na = _router(scores, bias_corr.reshape(1, E))
    zpad = ((0, 0), (0, IW - DR))
    d0 = jnp.pad(pos[:, 0].reshape(T // DR, DR), zpad)
    d1 = jnp.pad(pos[:, 1].reshape(T // DR, DR), zpad)
    gpad = ((0, 0), (0, IW - GR))
    g0 = jnp.pad(pos[:, 0].reshape(T // GR, GR), gpad)
    g1 = jnp.pad(pos[:, 1].reshape(T // GR, GR), gpad)

    x_sorted = _sc_dispatch(x, d0, d1)
    shared = _shared(x, Ws_gate, Ws_up, Ws_down)
    y_sorted = _gmm(be.reshape(NBMAX), na.reshape(1), x_sorted,
                    W_gate, W_up, W_down)
    y0, y1 = _sc_gather(y_sorted, g0, g1)
    return _epilogue(shared, y0, y1, w)
